# baseline (speedup 1.0000x reference)
import math

import jax
import jax.numpy as jnp
from jax import lax
from jax.experimental import pallas as pl
from jax.experimental.pallas import tpu as pltpu
from jax.experimental.pallas import tpu_sc as plsc

VOCAB = 1000000
DMODEL = 64
ROWS = 4096
COLS = 200
B = ROWS * COLS
NC = 2
NS = 16
NW = NC * NS
BPW = B // NW
SR = 4
C = SR * COLS
NCHUNK = BPW // C
NPAIR = NCHUNK // 2
SCALE = math.sqrt(DMODEL)
GATHERS = [(0, 128), (128, 128), (256, 128), (384, 128),
           (512, 128), (640, 128), (768, 32)]


def _body(w_hbm, x_hbm, out_hbm, idx_v, rows_v, gsem, osem):
    wid = lax.axis_index("s") * NC + lax.axis_index("c")
    base = wid * BPW
    row0 = wid * (ROWS // NW)

    def fire(slot, ci):
        off = base + ci * C
        pltpu.sync_copy(x_hbm.at[pl.ds(off, C)], idx_v.at[slot])
        for o, g in GATHERS:
            pltpu.async_copy(
                w_hbm.at[idx_v.at[slot, pl.ds(o, g)]],
                rows_v.at[slot, pl.ds(o, g)],
                gsem.at[slot],
            )

    def wait_gathers(slot):
        for o, g in GATHERS:
            pltpu.make_async_copy(
                w_hbm.at[idx_v.at[slot, pl.ds(o, g)]],
                rows_v.at[slot, pl.ds(o, g)],
                gsem.at[slot],
            ).wait()

    def write(slot, ci):
        r0 = row0 + ci * SR
        for si in range(SR):
            pltpu.async_copy(
                rows_v.at[slot, pl.ds(si * COLS, COLS)],
                out_hbm.at[r0 + si],
                osem.at[slot],
            )

    def wait_write(slot):
        for si in range(SR):
            pltpu.make_async_copy(
                rows_v.at[slot, pl.ds(si * COLS, COLS)],
                out_hbm.at[row0 + si],
                osem.at[slot],
            ).wait()

    fire(0, 0)

    @pl.loop(0, NPAIR)
    def pair(p):
        ci0 = p * 2
        wait_gathers(0)

        @pl.when(p > 0)
        def _():
            wait_write(1)

        fire(1, ci0 + 1)
        write(0, ci0)
        wait_gathers(1)

        @pl.when(p < NPAIR - 1)
        def _():
            wait_write(0)
            fire(0, ci0 + 2)

        write(1, ci0 + 1)

    wait_write(0)
    wait_write(1)


@jax.jit
def _embed(x_flat, W):
    fn = pl.kernel(
        _body,
        out_type=jax.ShapeDtypeStruct((ROWS, COLS, DMODEL), jnp.bfloat16),
        mesh=plsc.VectorSubcoreMesh(core_axis_name="c", subcore_axis_name="s"),
        scratch_types=[
            pltpu.VMEM((2, C), jnp.int32),
            pltpu.VMEM((2, C, DMODEL), jnp.bfloat16),
            pltpu.SemaphoreType.DMA((2,)),
            pltpu.SemaphoreType.DMA((2,)),
        ],
        compiler_params=pltpu.CompilerParams(use_tc_tiling_on_sc=False),
    )
    return fn(W, x_flat)


def kernel(x, W):
    out_bf = _embed(x.reshape(B), W.astype(jnp.bfloat16))
    return out_bf.astype(jnp.float32) * jnp.float32(SCALE)


# speedup vs baseline: 1.4577x; 1.4577x over previous
import math

import jax
import jax.numpy as jnp
from jax import lax
from jax.experimental import pallas as pl
from jax.experimental.pallas import tpu as pltpu
from jax.experimental.pallas import tpu_sc as plsc

VOCAB = 1000000
DMODEL = 64
ROWS = 4096
COLS = 200
B = ROWS * COLS
NC = 2
NS = 16
NW = NC * NS
BPW = B // NW
SR = 4
C = SR * COLS
NCHUNK = BPW // C
NPAIR = NCHUNK // 2
SCALE = math.sqrt(DMODEL)
GATHERS = [(0, 128), (128, 128), (256, 128), (384, 128),
           (512, 128), (640, 128), (768, 32)]


def _body(w_hbm, x_hbm, out_hbm, idx_v, rows_v, gsem, osem):
    wid = lax.axis_index("s") * NC + lax.axis_index("c")
    base = wid * BPW
    row0 = wid * (ROWS // NW)

    def fire(slot, ci):
        off = base + ci * C
        pltpu.sync_copy(x_hbm.at[pl.ds(off, C)], idx_v.at[slot])
        for o, g in GATHERS:
            pltpu.async_copy(
                w_hbm.at[idx_v.at[slot, pl.ds(o, g)]],
                rows_v.at[slot, pl.ds(o, g)],
                gsem.at[slot],
            )

    def wait_gathers(slot):
        for o, g in GATHERS:
            pltpu.make_async_copy(
                w_hbm.at[idx_v.at[slot, pl.ds(o, g)]],
                rows_v.at[slot, pl.ds(o, g)],
                gsem.at[slot],
            ).wait()

    def scale(slot):
        @pl.loop(0, C, unroll=4)
        def srow(i):
            for j in range(DMODEL // 16):
                sl = pl.ds(j * 16, 16)
                rows_v[slot, i, sl] = rows_v[slot, i, sl] * SCALE

    def write(slot, ci):
        r0 = row0 + ci * SR
        for si in range(SR):
            pltpu.async_copy(
                rows_v.at[slot, pl.ds(si * COLS, COLS)],
                out_hbm.at[r0 + si],
                osem.at[slot],
            )

    def wait_write(slot):
        for si in range(SR):
            pltpu.make_async_copy(
                rows_v.at[slot, pl.ds(si * COLS, COLS)],
                out_hbm.at[row0 + si],
                osem.at[slot],
            ).wait()

    fire(0, 0)

    @pl.loop(0, NPAIR)
    def pair(p):
        ci0 = p * 2
        wait_gathers(0)

        @pl.when(p > 0)
        def _():
            wait_write(1)

        fire(1, ci0 + 1)
        scale(0)
        write(0, ci0)
        wait_gathers(1)

        @pl.when(p < NPAIR - 1)
        def _():
            wait_write(0)
            fire(0, ci0 + 2)

        scale(1)
        write(1, ci0 + 1)

    wait_write(0)
    wait_write(1)


@jax.jit
def _embed(x_flat, W):
    fn = pl.kernel(
        _body,
        out_type=jax.ShapeDtypeStruct((ROWS, COLS, DMODEL), jnp.float32),
        mesh=plsc.VectorSubcoreMesh(core_axis_name="c", subcore_axis_name="s"),
        scratch_types=[
            pltpu.VMEM((2, C), jnp.int32),
            pltpu.VMEM((2, C, DMODEL), jnp.float32),
            pltpu.SemaphoreType.DMA((2,)),
            pltpu.SemaphoreType.DMA((2,)),
        ],
        compiler_params=pltpu.CompilerParams(use_tc_tiling_on_sc=False),
    )
    return fn(W, x_flat)


def kernel(x, W):
    return _embed(x.reshape(B), W)


# speedup vs baseline: 1.4602x; 1.0017x over previous
import functools
import math

import jax
import jax.numpy as jnp
from jax import lax
from jax.experimental import pallas as pl
from jax.experimental.pallas import tpu as pltpu
from jax.experimental.pallas import tpu_sc as plsc
from jax.experimental.layout import Layout, Format

VOCAB = 1000000
DMODEL = 64
ROWS = 4096
COLS = 200
B = ROWS * COLS
NC = 2
NS = 16
NW = NC * NS
BPW = B // NW
SR = 4
C = SR * COLS
NCHUNK = BPW // C
NPAIR = NCHUNK // 2
SCALE = math.sqrt(DMODEL)
GATHERS = [(0, 128), (128, 128), (256, 128), (384, 128),
           (512, 128), (640, 128), (768, 32)]


def _body(w_hbm, x_hbm, out_hbm, idx_v, rows_v, gsem, osem):
    wid = lax.axis_index("s") * NC + lax.axis_index("c")
    base = wid * BPW
    row0 = wid * (ROWS // NW)

    def fire(slot, ci):
        off = base + ci * C
        pltpu.sync_copy(x_hbm.at[pl.ds(off, C)], idx_v.at[slot])
        for o, g in GATHERS:
            pltpu.async_copy(
                w_hbm.at[idx_v.at[slot, pl.ds(o, g)]],
                rows_v.at[slot, pl.ds(o, g)],
                gsem.at[slot],
            )

    def wait_gathers(slot):
        for o, g in GATHERS:
            pltpu.make_async_copy(
                w_hbm.at[idx_v.at[slot, pl.ds(o, g)]],
                rows_v.at[slot, pl.ds(o, g)],
                gsem.at[slot],
            ).wait()

    def scale(slot):
        @pl.loop(0, C, unroll=4)
        def srow(i):
            for j in range(DMODEL // 16):
                sl = pl.ds(j * 16, 16)
                rows_v[slot, i, sl] = rows_v[slot, i, sl] * SCALE

    def write(slot, ci):
        r0 = row0 + ci * SR
        for si in range(SR):
            pltpu.async_copy(
                rows_v.at[slot, pl.ds(si * COLS, COLS)],
                out_hbm.at[r0 + si],
                osem.at[slot],
            )

    def wait_write(slot):
        for si in range(SR):
            pltpu.make_async_copy(
                rows_v.at[slot, pl.ds(si * COLS, COLS)],
                out_hbm.at[row0 + si],
                osem.at[slot],
            ).wait()

    fire(0, 0)

    @pl.loop(0, NPAIR)
    def pair(p):
        ci0 = p * 2
        wait_gathers(0)

        @pl.when(p > 0)
        def _():
            wait_write(1)

        fire(1, ci0 + 1)
        scale(0)
        write(0, ci0)
        wait_gathers(1)

        @pl.when(p < NPAIR - 1)
        def _():
            wait_write(0)
            fire(0, ci0 + 2)

        scale(1)
        write(1, ci0 + 1)

    wait_write(0)
    wait_write(1)


def _out_fmt():
    return Format(Layout(major_to_minor=(0, 1, 2), tiling=((8, 128),)),
                  jax.sharding.SingleDeviceSharding(jax.devices()[0]))


@functools.cache
def _embed_fn():
    return jax.jit(_embed, out_shardings=_out_fmt())


def _embed(x_flat, W):
    fn = pl.kernel(
        _body,
        out_type=jax.ShapeDtypeStruct((ROWS, COLS, DMODEL), jnp.float32),
        mesh=plsc.VectorSubcoreMesh(core_axis_name="c", subcore_axis_name="s"),
        scratch_types=[
            pltpu.VMEM((2, C), jnp.int32),
            pltpu.VMEM((2, C, DMODEL), jnp.float32),
            pltpu.SemaphoreType.DMA((2,)),
            pltpu.SemaphoreType.DMA((2,)),
        ],
        compiler_params=pltpu.CompilerParams(use_tc_tiling_on_sc=False),
    )
    return fn(W, x_flat)


def kernel(x, W):
    return _embed_fn()(x.reshape(B), W)
